# Initial kernel scaffold; baseline (speedup 1.0000x reference)
import jax
import jax.numpy as jnp
from jax.experimental import pallas as pl


def kernel(x, edge_index, W, b):
    raise NotImplementedError("write your pallas kernel here")


# speedup vs baseline: 2.9787x; 2.9787x over previous
import jax
import jax.numpy as jnp
from jax import lax
from jax.experimental import pallas as pl

N = 10000
D = 64
BM = 400


def _h_body(dis_ref, p_ref, w_ref, b_ref, h_ref):
    pre = dis_ref[...] * p_ref[...]
    h = jnp.dot(pre, w_ref[...], preferred_element_type=jnp.float32) + b_ref[...]
    h_ref[...] = jnp.maximum(h, 0.0)


def _mm_body(hi_ref, hall_ref, out_ref):
    out_ref[...] = lax.dot_general(
        hi_ref[...], hall_ref[...], (((1,), (1,)), ((), ())),
        preferred_element_type=jnp.float32)


def kernel(x, edge_index, W, b):
    src = edge_index[0].astype(jnp.int32)
    dst = edge_index[1].astype(jnp.int32)
    deg = jnp.zeros((N,), jnp.float32).at[dst].add(1.0) + 1.0
    dis = lax.rsqrt(deg)
    y = dis[:, None] * x
    p = y + jnp.zeros_like(y).at[dst].add(jnp.take(y, src, axis=0))

    h = pl.pallas_call(
        _h_body,
        out_shape=jax.ShapeDtypeStruct((N, D), jnp.float32),
    )(dis[:, None], p, W, b[None, :])

    adj = pl.pallas_call(
        _mm_body,
        grid=(N // BM,),
        in_specs=[
            pl.BlockSpec((BM, D), lambda i: (i, 0)),
            pl.BlockSpec((N, D), lambda i: (0, 0)),
        ],
        out_specs=pl.BlockSpec((BM, N), lambda i: (i, 0)),
        out_shape=jax.ShapeDtypeStruct((N, N), jnp.float32),
    )(h, h)
    return adj


# speedup vs baseline: 31.7876x; 10.6715x over previous
import functools

import jax
import jax.numpy as jnp
from jax import lax
from jax.experimental import pallas as pl
from jax.experimental.pallas import tpu as pltpu
from jax.experimental.pallas import tpu_sc as plsc

N = 10000
D = 64
NP = 10240
E = 640000
NW = 32
CH = 128
NCH = 158
EPT = NCH * CH
EPAD = NW * EPT
TROWS = NP // 16

_mesh = plsc.VectorSubcoreMesh(core_axis_name="c", subcore_axis_name="s")
_sc_params = pltpu.CompilerParams(use_tc_tiling_on_sc=False,
                                  needs_layout_passes=False)


@functools.partial(
    pl.kernel,
    out_type=jax.ShapeDtypeStruct((NW, NP), jnp.float32),
    mesh=_mesh,
    compiler_params=_sc_params,
    scratch_types=[
        pltpu.VMEM((NCH, CH), jnp.int32),
        pltpu.VMEM((NP,), jnp.float32),
    ],
)
def _deg_kernel(dst_hbm, deg_out, dst_v, hist_v):
    c = lax.axis_index("c")
    s = lax.axis_index("s")
    wid = s * 2 + c

    pltpu.sync_copy(dst_hbm.at[wid], dst_v)

    def _fill_z(i, _):
        hist_v[pl.ds(i * 16, 16)] = jnp.zeros((16,), jnp.float32)
        return _
    lax.fori_loop(0, NP // 16, _fill_z, None)

    ones = jnp.full((16,), 1.0, jnp.float32)

    def _hist(r, _):
        for j in range(CH // 16):
            idx = dst_v[r, pl.ds(j * 16, 16)]
            plsc.addupdate_scatter(hist_v, [idx], ones)
        return _
    lax.fori_loop(0, NCH, _hist, None)

    pltpu.sync_copy(hist_v, deg_out.at[wid])


@functools.partial(
    pl.kernel,
    out_type=jax.ShapeDtypeStruct((2, NP, D), jnp.float32),
    mesh=_mesh,
    compiler_params=_sc_params,
    scratch_types=[
        pltpu.VMEM((NCH, CH), jnp.int32),
        pltpu.VMEM((NCH, CH), jnp.int32),
        pltpu.VMEM((CH, D), jnp.float32),
        pltpu.VMEM((CH, D), jnp.float32),
        pltpu.SemaphoreType.DMA,
        pltpu.SemaphoreType.DMA,
        pltpu.VMEM_SHARED((NP, D), jnp.float32),
    ],
)
def _agg_kernel(y_hbm, src_hbm, dst_hbm, p_out,
                src_v, dst_v, buf0, buf1, sem0, sem1, acc_sh):
    c = lax.axis_index("c")
    s = lax.axis_index("s")
    wid = s * 2 + c
    base = s * TROWS

    pltpu.sync_copy(src_hbm.at[wid], src_v)
    pltpu.sync_copy(dst_hbm.at[wid], dst_v)

    pltpu.sync_copy(y_hbm.at[pl.ds(base, TROWS)], acc_sh.at[pl.ds(base, TROWS)])
    plsc.subcore_barrier()

    pltpu.async_copy(y_hbm.at[src_v.at[0]], buf0, sem0)
    pltpu.async_copy(y_hbm.at[src_v.at[1]], buf1, sem1)

    def _body(pair, _):
        ch = pair * 2
        for k, buf, sem in ((0, buf0, sem0), (1, buf1, sem1)):
            cc = ch + k
            pltpu.make_async_copy(y_hbm.at[src_v.at[cc]], buf, sem).wait()
            pltpu.sync_copy(buf, acc_sh.at[dst_v.at[cc]], add=True)
            nxt = cc + 2

            @pl.when(nxt < NCH)
            def _():
                pltpu.async_copy(y_hbm.at[src_v.at[lax.min(nxt, NCH - 1)]],
                                 buf, sem)
        return _
    lax.fori_loop(0, NCH // 2, _body, None)

    plsc.subcore_barrier()
    pltpu.sync_copy(acc_sh.at[pl.ds(base, TROWS)], p_out.at[c, pl.ds(base, TROWS)])


def _y_body(dis_ref, x_ref, y_ref):
    y_ref[...] = dis_ref[...] * x_ref[...]


def _h_body(dis_ref, p_ref, y_ref, w_ref, b_ref, h_ref):
    q = p_ref[0] + p_ref[1] - y_ref[...]
    pre = dis_ref[...] * q
    h = jnp.dot(pre, w_ref[...], preferred_element_type=jnp.float32) + b_ref[...]
    h_ref[...] = jnp.maximum(h, 0.0)


def _mm_body(hi_ref, hall_ref, out_ref):
    out_ref[...] = lax.dot_general(
        hi_ref[...], hall_ref[...], (((1,), (1,)), ((), ())),
        preferred_element_type=jnp.float32)


BM = 400


def kernel(x, edge_index, W, b):
    ei = edge_index.astype(jnp.int32)
    src = jnp.concatenate([ei[0], jnp.zeros((EPAD - E,), jnp.int32)])
    dst = jnp.concatenate([ei[1], jnp.full((EPAD - E,), N, jnp.int32)])
    src3 = src.reshape(NW, NCH, CH)
    dst3 = dst.reshape(NW, NCH, CH)
    xp = jnp.pad(x, ((0, NP - N), (0, 0)))

    deg_p = _deg_kernel(dst3)
    dis = lax.rsqrt(jnp.sum(deg_p, axis=0) + 1.0)[:, None]

    y = pl.pallas_call(
        _y_body,
        out_shape=jax.ShapeDtypeStruct((NP, D), jnp.float32),
    )(dis, xp)

    p = _agg_kernel(y, src3, dst3)

    h = pl.pallas_call(
        _h_body,
        out_shape=jax.ShapeDtypeStruct((NP, D), jnp.float32),
    )(dis, p, y, W, b[None, :])

    adj = pl.pallas_call(
        _mm_body,
        grid=(N // BM,),
        in_specs=[
            pl.BlockSpec((BM, D), lambda i: (i, 0)),
            pl.BlockSpec((N, D), lambda i: (0, 0)),
        ],
        out_specs=pl.BlockSpec((BM, N), lambda i: (i, 0)),
        out_shape=jax.ShapeDtypeStruct((N, N), jnp.float32),
    )(h, h)
    return adj
